# Initial kernel scaffold; baseline (speedup 1.0000x reference)
import jax
import jax.numpy as jnp
from jax.experimental import pallas as pl


def kernel(x, edge_attr, atom_tables, bond_tables):
    raise NotImplementedError("write your pallas kernel here")


# speedup vs baseline: 1.7187x; 1.7187x over previous
import functools

import jax
import jax.numpy as jnp
from jax import lax
from jax.experimental import pallas as pl
from jax.experimental.pallas import tpu as pltpu
from jax.experimental.pallas import tpu_sc as plsc

HIDDEN = 64
N_NODES = 50000
N_EDGES = 800000

NC = 2
NS = 16
NW = NC * NS
L = 16

E_PER_W = N_EDGES // NW
E_CHUNK = 1000
E_CHUNKS = E_PER_W // E_CHUNK
E_GROUPS = (E_CHUNK + L - 1) // L
E_SUB = (128, 128, 128, 128, 128, 128, 128, 104)

V_PER_W = 1568
V_CHUNK = 392
V_CHUNKS = V_PER_W // V_CHUNK
V_GROUPS = (V_CHUNK + L - 1) // L
V_SUB = (128, 128, 128, 8)


def _fused_gather(table_hbm, idx_ref, rows_ref, sem, subsizes):
    descs = []
    off = 0
    for sz in subsizes:
        descs.append(
            pltpu.async_copy(
                table_hbm.at[idx_ref.at[pl.ds(off, sz)]],
                rows_ref.at[pl.ds(off, sz)],
                sem,
            )
        )
        off += sz
    for d in descs:
        d.wait()


def _sc_body(x_hbm, ea_hbm, fa_hbm, fb_hbm, fc_hbm, fe_hbm,
             node_hbm, edge_hbm,
             xbuf, ebuf, ia, ib, ic, eidx, rows_a, rows_b, rows_e, sem):
    cid = lax.axis_index("c")
    sid = lax.axis_index("s")
    wid = sid * NC + cid
    lanes = lax.iota(jnp.int32, L)

    node_base = jnp.minimum(wid * V_PER_W, N_NODES - V_PER_W)

    def node_chunk(k, carry):
        base = node_base + k * V_CHUNK
        pltpu.sync_copy(x_hbm.at[pl.ds(base * 9, V_CHUNK * 9)], xbuf)

        def grp(g, c2):
            r = jnp.minimum(g * L + lanes, V_CHUNK - 1) * 9
            cols = [plsc.load_gather(xbuf, [r + j]) for j in range(9)]
            va = cols[0] * 4 + cols[1]
            vb = (cols[2] * 12 + cols[3]) * 10 + cols[4]
            vc = ((cols[5] * 6 + cols[6]) * 2 + cols[7]) * 2 + cols[8]
            start = pl.multiple_of(g * L, L)
            ia[pl.ds(start, L)] = va
            ib[pl.ds(start, L)] = vb
            ic[pl.ds(start, L)] = vc
            return c2

        lax.fori_loop(0, V_GROUPS, grp, 0)

        _fused_gather(fa_hbm, ia, rows_a, sem, V_SUB)
        _fused_gather(fb_hbm, ib, rows_b, sem, V_SUB)

        def add_row(r, c2):
            for c in range(HIDDEN // L):
                sl = (r, pl.ds(c * L, L))
                rows_a[sl] = rows_a[sl] + rows_b[sl]
            return c2

        lax.fori_loop(0, V_CHUNK, add_row, 0)

        _fused_gather(fc_hbm, ic, rows_b, sem, V_SUB)
        lax.fori_loop(0, V_CHUNK, add_row, 0)

        pltpu.sync_copy(rows_a, node_hbm.at[pl.ds(base, V_CHUNK)])
        return carry

    lax.fori_loop(0, V_CHUNKS, node_chunk, 0)

    e_base0 = wid * E_PER_W

    def edge_chunk(k, carry):
        base = e_base0 + k * E_CHUNK
        pltpu.sync_copy(ea_hbm.at[pl.ds(base * 3, E_CHUNK * 3)], ebuf)

        def grp(g, c2):
            r = jnp.minimum(g * L + lanes, E_CHUNK - 1) * 3
            e0 = plsc.load_gather(ebuf, [r])
            e1 = plsc.load_gather(ebuf, [r + 1])
            e2 = plsc.load_gather(ebuf, [r + 2])
            start = pl.multiple_of(g * L, L)
            eidx[pl.ds(start, L)] = (e0 * 6 + e1) * 2 + e2
            return c2

        lax.fori_loop(0, E_GROUPS, grp, 0)

        _fused_gather(fe_hbm, eidx, rows_e, sem, E_SUB)
        pltpu.sync_copy(rows_e, edge_hbm.at[pl.ds(base, E_CHUNK)])
        return carry

    lax.fori_loop(0, E_CHUNKS, edge_chunk, 0)


_sc_call = pl.kernel(
    _sc_body,
    out_type=(
        jax.ShapeDtypeStruct((N_NODES, HIDDEN), jnp.float32),
        jax.ShapeDtypeStruct((N_EDGES, HIDDEN), jnp.float32),
    ),
    mesh=plsc.VectorSubcoreMesh(core_axis_name="c", subcore_axis_name="s"),
    compiler_params=pltpu.CompilerParams(
        needs_layout_passes=False, use_tc_tiling_on_sc=False),
    scratch_types=[
        pltpu.VMEM((V_CHUNK * 9,), jnp.int32),
        pltpu.VMEM((E_CHUNK * 3,), jnp.int32),
        pltpu.VMEM((V_GROUPS * L,), jnp.int32),
        pltpu.VMEM((V_GROUPS * L,), jnp.int32),
        pltpu.VMEM((V_GROUPS * L,), jnp.int32),
        pltpu.VMEM((E_GROUPS * L,), jnp.int32),
        pltpu.VMEM((V_CHUNK, HIDDEN), jnp.float32),
        pltpu.VMEM((V_CHUNK, HIDDEN), jnp.float32),
        pltpu.VMEM((E_CHUNK, HIDDEN), jnp.float32),
        pltpu.SemaphoreType.DMA,
    ],
)


def kernel(x, edge_attr, atom_tables, bond_tables):
    x = x.astype(jnp.int32).reshape(-1)
    ea = edge_attr.astype(jnp.int32).reshape(-1)
    t = [a.astype(jnp.float32) for a in atom_tables]
    b = [a.astype(jnp.float32) for a in bond_tables]
    fa = (t[0][:, None] + t[1][None, :]).reshape(119 * 4, HIDDEN)
    fb = (t[2][:, None, None] + t[3][None, :, None]
          + t[4][None, None, :]).reshape(12 * 12 * 10, HIDDEN)
    fc = (t[5][:, None, None, None] + t[6][None, :, None, None]
          + t[7][None, None, :, None]
          + t[8][None, None, None, :]).reshape(6 * 6 * 2 * 2, HIDDEN)
    fe = (b[0][:, None, None] + b[1][None, :, None]
          + b[2][None, None, :]).reshape(5 * 6 * 2, HIDDEN)
    node_emb, edge_emb = _sc_call(x, ea, fa, fb, fc, fe)
    return (node_emb, edge_emb)
